# Initial kernel scaffold; baseline (speedup 1.0000x reference)
import jax
import jax.numpy as jnp
from jax.experimental import pallas as pl


def kernel(x, w1, b1, w2, b2, w3, b3, w4, b4):
    raise NotImplementedError("write your pallas kernel here")


# speedup vs baseline: 3.0009x; 3.0009x over previous
import functools

import jax
import jax.numpy as jnp
from jax.experimental import pallas as pl
from jax.experimental.pallas import tpu as pltpu

_LANE = 128


def _msfeat_fused_kernel(bias_ref, w1_ref, w2_ref, w3_ref, w4_ref,
                         x_ref,
                         out_ref,
                         *, C, Cin, H, W, Bt):
    LS = Bt * W

    col = jax.lax.broadcasted_iota(jnp.int32, (1, LS), 1) % W

    def shifted(plane, dx):
        if dx == 0:
            return plane
        rolled = pltpu.roll(plane, (-dx) % LS, axis=1)
        ok = (col < W - dx) if dx > 0 else (col >= -dx)
        return jnp.where(ok, rolled, 0.0)

    def rowwin(plane, dy):
        if dy == 0:
            return plane
        z = jnp.zeros((abs(dy), LS), jnp.float32)
        if dy > 0:
            return jnp.concatenate([plane[dy:, :], z], axis=0)
        return jnp.concatenate([z, plane[:H + dy, :]], axis=0)

    def conv(shifts, w_ref, n_in, ob, dil):
        acc = [jnp.full((H, LS), bias_ref[ob + oc], jnp.float32)
               for oc in range(C)]
        for ic in range(n_in):
            for kx in range(3):
                sp = shifts[ic][kx]
                for ky in range(3):
                    win = rowwin(sp, dil * (ky - 1))
                    base = ic * 9 + ky * 3 + kx
                    for oc in range(C):
                        acc[oc] = acc[oc] + w_ref[oc * n_in * 9 + base] * win
        return [jnp.maximum(a, 0.0) for a in acc]

    xs = [jnp.concatenate([x_ref[b, ic] for b in range(Bt)], axis=1)
          for ic in range(Cin)]

    sh1 = [[shifted(p, 1 * (kx - 1)) for kx in range(3)] for p in xs]
    sh2 = [[shifted(p, 2 * (kx - 1)) for kx in range(3)] for p in xs]
    c1 = conv(sh1, w1_ref, Cin, 0 * C, 1)
    c2 = conv(sh2, w2_ref, Cin, 1 * C, 2)
    sh3 = [[shifted(p, 1 * (kx - 1)) for kx in range(3)] for p in c2]
    sh4 = [[shifted(p, 2 * (kx - 1)) for kx in range(3)] for p in c1]
    c3 = conv(sh3, w3_ref, C, 2 * C, 1)
    c4 = conv(sh4, w4_ref, C, 3 * C, 2)

    for idx, c in enumerate(c1 + c2 + c3 + c4):
        for b in range(Bt):
            out_ref[b, idx, :, :] = c[:, b * W:(b + 1) * W]


def _pick_bt(B, W):
    cands = [bt for bt in range(1, 33)
             if (bt * W) % _LANE == 0 and bt * W <= 1024]
    if not cands:
        return 1
    div = [bt for bt in cands if B % bt == 0]
    pool = div if div else cands
    return min(pool, key=lambda bt: abs(bt * W - 512))


def kernel(x, w1, b1, w2, b2, w3, b3, w4, b4):
    B, Cin, H, W = x.shape
    C = w1.shape[0]
    Bt = _pick_bt(B, W)
    NB = -(-B // Bt)
    Bp = NB * Bt

    xf = x.astype(jnp.float32)
    if Bp != B:
        xf = jnp.concatenate(
            [xf, jnp.zeros((Bp - B, Cin, H, W), jnp.float32)], axis=0)

    bias = jnp.concatenate([b1, b2, b3, b4]).astype(jnp.float32)
    w1f = w1.reshape(-1).astype(jnp.float32)
    w2f = w2.reshape(-1).astype(jnp.float32)
    w3f = w3.reshape(-1).astype(jnp.float32)
    w4f = w4.reshape(-1).astype(jnp.float32)

    fn = functools.partial(_msfeat_fused_kernel,
                           C=C, Cin=Cin, H=H, W=W, Bt=Bt)

    out = pl.pallas_call(
        fn,
        out_shape=jax.ShapeDtypeStruct((Bp, 4 * C, H, W), jnp.float32),
        grid_spec=pltpu.PrefetchScalarGridSpec(
            num_scalar_prefetch=5,
            grid=(NB,),
            in_specs=[
                pl.BlockSpec((Bt, Cin, H, W), lambda i, *_: (i, 0, 0, 0)),
            ],
            out_specs=pl.BlockSpec((Bt, 4 * C, H, W),
                                   lambda i, *_: (i, 0, 0, 0)),
        ),
        compiler_params=pltpu.CompilerParams(
            dimension_semantics=("parallel",),
            vmem_limit_bytes=64 << 20),
    )(bias, w1f, w2f, w3f, w4f, xf)
    return out[:B] if Bp != B else out


# speedup vs baseline: 6.6141x; 2.2041x over previous
import functools

import jax
import jax.numpy as jnp
from jax.experimental import pallas as pl
from jax.experimental.pallas import tpu as pltpu

_LANE = 128


def _msfeat_kernel(bias_ref, w1_ref, w2_ref, w3_ref, w4_ref,
                   x_ref,
                   o_ref,
                   xs_ref,
                   *, C, Cin, H, W, Bb, H2):
    j = pl.program_id(1)
    HW = H2 + 4

    @pl.when(j == 0)
    def _fill():
        z = jnp.zeros((4, W, Bb), jnp.float32)
        for ic in range(Cin):
            xs_ref[ic, 0:4] = z
            xs_ref[ic, 4:4 + H] = x_ref[ic]
            xs_ref[ic, 4 + H:8 + H] = z

    base = j * H2

    r = jax.lax.broadcasted_iota(jnp.int32, (HW, 1, 1), 0)
    g = base - 2 + r
    vmask = ((g >= 0) & (g < H)).astype(jnp.float32)

    def wshift(p, dx, nrows):
        if dx == 0:
            return p
        z = jnp.zeros((nrows, abs(dx), Bb), jnp.float32)
        if dx > 0:
            return jnp.concatenate([p[:, dx:, :], z], axis=1)
        return jnp.concatenate([z, p[:, :W + dx, :]], axis=1)

    cs = []
    for w_ref, ob, dil in ((w1_ref, 0, 1), (w2_ref, C, 2)):
        accs = [None] * C
        for ic in range(Cin):
            xw = xs_ref[ic, pl.ds(base, HW + 4)]
            for kx in range(3):
                s = wshift(xw, dil * (kx - 1), HW + 4)
                for ky in range(3):
                    dy = dil * (ky - 1)
                    win = s[2 + dy:2 + dy + HW]
                    for oc in range(C):
                        wv = w_ref[(oc * Cin + ic) * 9 + ky * 3 + kx]
                        t = win * wv
                        accs[oc] = t if accs[oc] is None else accs[oc] + t
        for oc in range(C):
            c = jnp.maximum(accs[oc] + bias_ref[ob + oc], 0.0) * vmask
            o_ref[ob + oc] = c[2:2 + H2]
            cs.append(c)

    for w_ref, ob, dil, s0 in ((w3_ref, 2 * C, 1, C), (w4_ref, 3 * C, 2, 0)):
        accs = [None] * C
        for ic in range(C):
            p = cs[s0 + ic]
            for kx in range(3):
                s = wshift(p, dil * (kx - 1), HW)
                for ky in range(3):
                    dy = dil * (ky - 1)
                    win = s[2 + dy:2 + dy + H2]
                    for oc in range(C):
                        wv = w_ref[(oc * C + ic) * 9 + ky * 3 + kx]
                        t = win * wv
                        accs[oc] = t if accs[oc] is None else accs[oc] + t
        for oc in range(C):
            o_ref[ob + oc] = jnp.maximum(accs[oc] + bias_ref[ob + oc], 0.0)


def kernel(x, w1, b1, w2, b2, w3, b3, w4, b4):
    B, Cin, H, W = x.shape
    C = w1.shape[0]

    Bb = _LANE
    NB = -(-B // Bb)
    Bp = NB * Bb
    H2 = H // 2

    xf = x.astype(jnp.float32)
    if Bp != B:
        xf = jnp.concatenate(
            [xf, jnp.zeros((Bp - B, Cin, H, W), jnp.float32)], axis=0)

    xt = jnp.transpose(xf, (1, 2, 3, 0))

    bias = jnp.concatenate([b1, b2, b3, b4]).astype(jnp.float32)
    w1f = w1.reshape(-1).astype(jnp.float32)
    w2f = w2.reshape(-1).astype(jnp.float32)
    w3f = w3.reshape(-1).astype(jnp.float32)
    w4f = w4.reshape(-1).astype(jnp.float32)

    fn = functools.partial(_msfeat_kernel,
                           C=C, Cin=Cin, H=H, W=W, Bb=Bb, H2=H2)

    ot = pl.pallas_call(
        fn,
        out_shape=jax.ShapeDtypeStruct((4 * C, H, W, Bp), jnp.float32),
        grid_spec=pltpu.PrefetchScalarGridSpec(
            num_scalar_prefetch=5,
            grid=(NB, 2),
            in_specs=[
                pl.BlockSpec((Cin, H, W, Bb), lambda i, j, *_: (0, 0, 0, i)),
            ],
            out_specs=pl.BlockSpec((4 * C, H2, W, Bb),
                                   lambda i, j, *_: (0, j, 0, i)),
            scratch_shapes=[pltpu.VMEM((Cin, H + 8, W, Bb), jnp.float32)],
        ),
        compiler_params=pltpu.CompilerParams(
            dimension_semantics=("parallel", "arbitrary"),
            vmem_limit_bytes=64 << 20),
    )(bias, w1f, w2f, w3f, w4f, xt)

    out = jnp.transpose(ot, (3, 0, 1, 2))
    return out[:B] if Bp != B else out


# speedup vs baseline: 6.6887x; 1.0113x over previous
import functools

import jax
import jax.numpy as jnp
from jax.experimental import pallas as pl
from jax.experimental.pallas import tpu as pltpu

_LANE = 128


def _msfeat_kernel(w1_ref, b1_ref, w2_ref, b2_ref, w3_ref, b3_ref,
                   w4_ref, b4_ref,
                   x_ref,
                   o_ref,
                   xs_ref,
                   *, C, Cin, H, W, Bb, H2):
    j = pl.program_id(1)
    HW = H2 + 4

    @pl.when(j == 0)
    def _fill():
        z = jnp.zeros((4, W, Bb), jnp.float32)
        for ic in range(Cin):
            xs_ref[ic, 0:4] = z
            xs_ref[ic, 4:4 + H] = x_ref[ic]
            xs_ref[ic, 4 + H:8 + H] = z

    base = j * H2

    r = jax.lax.broadcasted_iota(jnp.int32, (HW, 1, 1), 0)
    g = base - 2 + r
    vmask = ((g >= 0) & (g < H)).astype(jnp.float32)

    def wshift(p, dx, nrows):
        if dx == 0:
            return p
        z = jnp.zeros((nrows, abs(dx), Bb), jnp.float32)
        if dx > 0:
            return jnp.concatenate([p[:, dx:, :], z], axis=1)
        return jnp.concatenate([z, p[:, :W + dx, :]], axis=1)

    def direct_conv(w_ref, b_ref, ob, dil):
        accs = [None] * C
        for ic in range(Cin):
            xw = xs_ref[ic, pl.ds(base, HW + 4)]
            for kx in range(3):
                s = wshift(xw, dil * (kx - 1), HW + 4)
                for ky in range(3):
                    dy = dil * (ky - 1)
                    win = s[2 + dy:2 + dy + HW]
                    for oc in range(C):
                        t = win * w_ref[oc, ic, ky, kx]
                        accs[oc] = t if accs[oc] is None else accs[oc] + t
        cs = []
        for oc in range(C):
            c = jnp.maximum(accs[oc] + b_ref[oc], 0.0) * vmask
            o_ref[ob + oc] = c[2:2 + H2]
            cs.append(c)
        return cs

    def chained_conv(cs, w_ref, b_ref, ob, dil):
        accs = [None] * C
        for ic in range(C):
            for kx in range(3):
                s = wshift(cs[ic], dil * (kx - 1), HW)
                for ky in range(3):
                    dy = dil * (ky - 1)
                    win = s[2 + dy:2 + dy + H2]
                    for oc in range(C):
                        t = win * w_ref[oc, ic, ky, kx]
                        accs[oc] = t if accs[oc] is None else accs[oc] + t
        for oc in range(C):
            o_ref[ob + oc] = jnp.maximum(accs[oc] + b_ref[oc], 0.0)

    c1 = direct_conv(w1_ref, b1_ref, 0 * C, 1)
    chained_conv(c1, w4_ref, b4_ref, 3 * C, 2)
    c2 = direct_conv(w2_ref, b2_ref, 1 * C, 2)
    chained_conv(c2, w3_ref, b3_ref, 2 * C, 1)


def kernel(x, w1, b1, w2, b2, w3, b3, w4, b4):
    B, Cin, H, W = x.shape
    C = w1.shape[0]

    Bb = _LANE
    NB = -(-B // Bb)
    Bp = NB * Bb
    H2 = H // 2

    xf = x.astype(jnp.float32)
    if Bp != B:
        xf = jnp.concatenate(
            [xf, jnp.zeros((Bp - B, Cin, H, W), jnp.float32)], axis=0)

    xt = jnp.transpose(xf, (1, 2, 3, 0))

    fn = functools.partial(_msfeat_kernel,
                           C=C, Cin=Cin, H=H, W=W, Bb=Bb, H2=H2)

    ot = pl.pallas_call(
        fn,
        out_shape=jax.ShapeDtypeStruct((4 * C, H, W, Bp), jnp.float32),
        grid_spec=pltpu.PrefetchScalarGridSpec(
            num_scalar_prefetch=8,
            grid=(NB, 2),
            in_specs=[
                pl.BlockSpec((Cin, H, W, Bb), lambda i, j, *_: (0, 0, 0, i)),
            ],
            out_specs=pl.BlockSpec((4 * C, H2, W, Bb),
                                   lambda i, j, *_: (0, j, 0, i)),
            scratch_shapes=[pltpu.VMEM((Cin, H + 8, W, Bb), jnp.float32)],
        ),
        compiler_params=pltpu.CompilerParams(
            dimension_semantics=("parallel", "arbitrary"),
            vmem_limit_bytes=64 << 20),
    )(w1.astype(jnp.float32), b1.astype(jnp.float32),
      w2.astype(jnp.float32), b2.astype(jnp.float32),
      w3.astype(jnp.float32), b3.astype(jnp.float32),
      w4.astype(jnp.float32), b4.astype(jnp.float32), xt)

    out = jnp.transpose(ot, (3, 0, 1, 2))
    return out[:B] if Bp != B else out


# speedup vs baseline: 7.2366x; 1.0819x over previous
import functools

import jax
import jax.numpy as jnp
from jax.experimental import pallas as pl
from jax.experimental.pallas import tpu as pltpu

_LANE = 128


def _msfeat_kernel(w1_ref, b1_ref, w2_ref, b2_ref, w3_ref, b3_ref,
                   w4_ref, b4_ref,
                   x_ref,
                   o_ref,
                   xs_ref,
                   *, C, Cin, H, W, Bb, H2):
    j = pl.program_id(1)
    HW = H2 + 4

    @pl.when(j == 0)
    def _fill():
        z = jnp.zeros((4, W, Bb), jnp.float32)
        for ic in range(Cin):
            xs_ref[ic, 0:4] = z
            xs_ref[ic, 4:4 + H] = x_ref[ic]
            xs_ref[ic, 4 + H:8 + H] = z

    base = j * H2

    r = jax.lax.broadcasted_iota(jnp.int32, (HW, 1, 1), 0)
    g = base - 2 + r
    vmask = ((g >= 0) & (g < H)).astype(jnp.float32)

    def wshift(p, dx, nrows):
        if dx == 0:
            return p
        z = jnp.zeros((nrows, abs(dx), Bb), jnp.float32)
        if dx > 0:
            return jnp.concatenate([p[:, dx:, :], z], axis=1)
        return jnp.concatenate([z, p[:, :W + dx, :]], axis=1)

    def direct_conv(w_ref, b_ref, ob, dil):
        cs = []
        for oc in range(C):
            Q = [None, None, None]
            for ic in range(Cin):
                xw = xs_ref[ic, pl.ds(base, HW + 4)]
                for kx in range(3):
                    for ky in range(3):
                        dy = dil * (ky - 1)
                        win = xw[2 + dy:2 + dy + HW]
                        t = win * w_ref[oc, ic, ky, kx]
                        Q[kx] = t if Q[kx] is None else Q[kx] + t
            acc = (Q[1] + b_ref[oc]) \
                + wshift(Q[0], -dil, HW) + wshift(Q[2], dil, HW)
            c = jnp.maximum(acc, 0.0) * vmask
            o_ref[ob + oc] = c[2:2 + H2]
            cs.append(c)
        return cs

    def chained_conv(cs, w_ref, b_ref, ob, dil):
        for oc in range(C):
            Q = [None, None, None]
            for ic in range(C):
                p = cs[ic]
                for kx in range(3):
                    for ky in range(3):
                        dy = dil * (ky - 1)
                        win = p[2 + dy:2 + dy + H2]
                        t = win * w_ref[oc, ic, ky, kx]
                        Q[kx] = t if Q[kx] is None else Q[kx] + t
            acc = (Q[1] + b_ref[oc]) \
                + wshift(Q[0], -dil, H2) + wshift(Q[2], dil, H2)
            o_ref[ob + oc] = jnp.maximum(acc, 0.0)

    c1 = direct_conv(w1_ref, b1_ref, 0 * C, 1)
    chained_conv(c1, w4_ref, b4_ref, 3 * C, 2)
    c2 = direct_conv(w2_ref, b2_ref, 1 * C, 2)
    chained_conv(c2, w3_ref, b3_ref, 2 * C, 1)


def kernel(x, w1, b1, w2, b2, w3, b3, w4, b4):
    B, Cin, H, W = x.shape
    C = w1.shape[0]

    Bb = _LANE
    NB = -(-B // Bb)
    Bp = NB * Bb
    H2 = H // 2

    xf = x.astype(jnp.float32)
    if Bp != B:
        xf = jnp.concatenate(
            [xf, jnp.zeros((Bp - B, Cin, H, W), jnp.float32)], axis=0)

    xt = jnp.transpose(xf, (1, 2, 3, 0))

    fn = functools.partial(_msfeat_kernel,
                           C=C, Cin=Cin, H=H, W=W, Bb=Bb, H2=H2)

    ot = pl.pallas_call(
        fn,
        out_shape=jax.ShapeDtypeStruct((4 * C, H, W, Bp), jnp.float32),
        grid_spec=pltpu.PrefetchScalarGridSpec(
            num_scalar_prefetch=8,
            grid=(NB, 2),
            in_specs=[
                pl.BlockSpec((Cin, H, W, Bb), lambda i, j, *_: (0, 0, 0, i)),
            ],
            out_specs=pl.BlockSpec((4 * C, H2, W, Bb),
                                   lambda i, j, *_: (0, j, 0, i)),
            scratch_shapes=[pltpu.VMEM((Cin, H + 8, W, Bb), jnp.float32)],
        ),
        compiler_params=pltpu.CompilerParams(
            dimension_semantics=("parallel", "arbitrary"),
            vmem_limit_bytes=64 << 20),
    )(w1.astype(jnp.float32), b1.astype(jnp.float32),
      w2.astype(jnp.float32), b2.astype(jnp.float32),
      w3.astype(jnp.float32), b3.astype(jnp.float32),
      w4.astype(jnp.float32), b4.astype(jnp.float32), xt)

    out = jnp.transpose(ot, (3, 0, 1, 2))
    return out[:B] if Bp != B else out


# speedup vs baseline: 7.3159x; 1.0110x over previous
import functools

import jax
import jax.numpy as jnp
from jax.experimental import pallas as pl
from jax.experimental.pallas import tpu as pltpu

_LANE = 128


def _msfeat_kernel(wb_ref,
                   x_ref,
                   o_ref,
                   xs_ref,
                   *, C, Cin, H, W, Bb, H2):
    j = pl.program_id(1)
    HW = H2 + 4

    @pl.when(j == 0)
    def _fill():
        z = jnp.zeros((4, W, Bb), jnp.float32)
        for ic in range(Cin):
            xs_ref[ic, 0:4] = z
            xs_ref[ic, 4:4 + H] = x_ref[ic]
            xs_ref[ic, 4 + H:8 + H] = z

    base = j * H2

    r = jax.lax.broadcasted_iota(jnp.int32, (HW, 1, 1), 0)
    g = base - 2 + r
    vmask = ((g >= 0) & (g < H)).astype(jnp.float32)

    def wshift(p, dx, nrows):
        if dx == 0:
            return p
        z = jnp.zeros((nrows, abs(dx), Bb), jnp.float32)
        if dx > 0:
            return jnp.concatenate([p[:, dx:, :], z], axis=1)
        return jnp.concatenate([z, p[:, :W + dx, :]], axis=1)

    nw = 9 * C * (2 * Cin + 2 * C)

    def direct_conv(woff, boff, ob, dil):
        cs = []
        for oc in range(C):
            Q = [None, None, None]
            for ic in range(Cin):
                xw = xs_ref[ic, pl.ds(base, HW + 4)]
                for kx in range(3):
                    for ky in range(3):
                        dy = dil * (ky - 1)
                        win = xw[2 + dy:2 + dy + HW]
                        t = win * wb_ref[woff + (oc * Cin + ic) * 9
                                         + ky * 3 + kx]
                        Q[kx] = t if Q[kx] is None else Q[kx] + t
            acc = (Q[1] + wb_ref[nw + boff + oc]) \
                + wshift(Q[0], -dil, HW) + wshift(Q[2], dil, HW)
            c = jnp.maximum(acc, 0.0) * vmask
            o_ref[ob + oc] = c[2:2 + H2]
            cs.append(c)
        return cs

    def chained_conv(cs, woff, boff, ob, dil):
        for oc in range(C):
            Q = [None, None, None]
            for ic in range(C):
                p = cs[ic]
                for kx in range(3):
                    for ky in range(3):
                        dy = dil * (ky - 1)
                        win = p[2 + dy:2 + dy + H2]
                        t = win * wb_ref[woff + (oc * C + ic) * 9
                                         + ky * 3 + kx]
                        Q[kx] = t if Q[kx] is None else Q[kx] + t
            acc = (Q[1] + wb_ref[nw + boff + oc]) \
                + wshift(Q[0], -dil, H2) + wshift(Q[2], dil, H2)
            o_ref[ob + oc] = jnp.maximum(acc, 0.0)

    n1 = 9 * C * Cin
    n3 = 9 * C * C
    c1 = direct_conv(0, 0 * C, 0 * C, 1)
    chained_conv(c1, 2 * n1 + n3, 3 * C, 3 * C, 2)
    c2 = direct_conv(n1, 1 * C, 1 * C, 2)
    chained_conv(c2, 2 * n1, 2 * C, 2 * C, 1)


def kernel(x, w1, b1, w2, b2, w3, b3, w4, b4):
    B, Cin, H, W = x.shape
    C = w1.shape[0]

    Bb = _LANE
    NB = -(-B // Bb)
    Bp = NB * Bb
    H2 = H // 2

    xf = x.astype(jnp.float32)
    if Bp != B:
        xf = jnp.concatenate(
            [xf, jnp.zeros((Bp - B, Cin, H, W), jnp.float32)], axis=0)

    xt = jnp.transpose(xf, (1, 2, 3, 0))

    wb = jnp.concatenate(
        [w1.reshape(-1), w2.reshape(-1), w3.reshape(-1), w4.reshape(-1),
         b1, b2, b3, b4]).astype(jnp.float32)

    fn = functools.partial(_msfeat_kernel,
                           C=C, Cin=Cin, H=H, W=W, Bb=Bb, H2=H2)

    ot = pl.pallas_call(
        fn,
        out_shape=jax.ShapeDtypeStruct((4 * C, H, W, Bp), jnp.float32),
        grid_spec=pltpu.PrefetchScalarGridSpec(
            num_scalar_prefetch=1,
            grid=(NB, 2),
            in_specs=[
                pl.BlockSpec((Cin, H, W, Bb), lambda i, j, *_: (0, 0, 0, i)),
            ],
            out_specs=pl.BlockSpec((4 * C, H2, W, Bb),
                                   lambda i, j, *_: (0, j, 0, i)),
            scratch_shapes=[pltpu.VMEM((Cin, H + 8, W, Bb), jnp.float32)],
        ),
        compiler_params=pltpu.CompilerParams(
            dimension_semantics=("parallel", "arbitrary"),
            vmem_limit_bytes=64 << 20),
    )(wb, xt)

    out = jnp.transpose(ot, (3, 0, 1, 2))
    return out[:B] if Bp != B else out


# speedup vs baseline: 7.8450x; 1.0723x over previous
import functools

import jax
import jax.numpy as jnp
from jax.experimental import pallas as pl
from jax.experimental.pallas import tpu as pltpu

_LANE = 128


def _msfeat_kernel(wb_ref,
                   x_ref,
                   o_ref,
                   cs_ref,
                   *, C, Cin, H, W, Bb, H2):
    j = pl.program_id(1)
    nw = 9 * C * (2 * Cin + 2 * C)
    n1 = 9 * C * Cin
    n3 = 9 * C * C

    def wshift(p, dx, nrows):
        if dx == 0:
            return p
        z = jnp.zeros((nrows, abs(dx), Bb), jnp.float32)
        if dx > 0:
            return jnp.concatenate([p[:, dx:, :], z], axis=1)
        return jnp.concatenate([z, p[:, :W + dx, :]], axis=1)

    def hwin(p, dy):
        if dy == 0:
            return p
        z = jnp.zeros((abs(dy), W, Bb), jnp.float32)
        if dy > 0:
            return jnp.concatenate([p[dy:], z], axis=0)
        return jnp.concatenate([z, p[:H + dy]], axis=0)

    def direct_conv(woff, boff, ob, dil):
        for oc in range(C):
            Q = [None, None, None]
            for ic in range(Cin):
                xv = x_ref[ic]
                for ky in range(3):
                    win = hwin(xv, dil * (ky - 1))
                    for kx in range(3):
                        t = win * wb_ref[woff + (oc * Cin + ic) * 9
                                         + ky * 3 + kx]
                        Q[kx] = t if Q[kx] is None else Q[kx] + t
            acc = (Q[1] + wb_ref[nw + boff + oc]) \
                + wshift(Q[0], -dil, H) + wshift(Q[2], dil, H)
            c = jnp.maximum(acc, 0.0)
            o_ref[ob + oc] = c[0:H2]
            cs_ref[ob + oc, 2:2 + H] = c
            cs_ref[ob + oc, 0:2] = jnp.zeros((2, W, Bb), jnp.float32)
            cs_ref[ob + oc, 2 + H:4 + H] = jnp.zeros((2, W, Bb), jnp.float32)

    def chained_conv(src, woff, boff, ob, dil, h0):
        for oc in range(C):
            Q = [None, None, None]
            for ic in range(C):
                for kx in range(3):
                    for ky in range(3):
                        dy = dil * (ky - 1)
                        win = cs_ref[src + ic, 2 + dy + h0:2 + dy + h0 + H2]
                        t = win * wb_ref[woff + (oc * C + ic) * 9
                                         + ky * 3 + kx]
                        Q[kx] = t if Q[kx] is None else Q[kx] + t
            acc = (Q[1] + wb_ref[nw + boff + oc]) \
                + wshift(Q[0], -dil, H2) + wshift(Q[2], dil, H2)
            o_ref[ob + oc] = jnp.maximum(acc, 0.0)

    @pl.when(j == 0)
    def _top():
        direct_conv(0, 0 * C, 0 * C, 1)
        chained_conv(0 * C, 2 * n1 + n3, 3 * C, 3 * C, 2, 0)
        direct_conv(n1, 1 * C, 1 * C, 2)
        chained_conv(1 * C, 2 * n1, 2 * C, 2 * C, 1, 0)

    @pl.when(j == 1)
    def _bottom():
        for ch in range(2 * C):
            o_ref[ch] = cs_ref[ch, 2 + H2:2 + H]
        chained_conv(0 * C, 2 * n1 + n3, 3 * C, 3 * C, 2, H2)
        chained_conv(1 * C, 2 * n1, 2 * C, 2 * C, 1, H2)


def kernel(x, w1, b1, w2, b2, w3, b3, w4, b4):
    B, Cin, H, W = x.shape
    C = w1.shape[0]

    Bb = _LANE
    NB = -(-B // Bb)
    Bp = NB * Bb
    H2 = H // 2

    xf = x.astype(jnp.float32)
    if Bp != B:
        xf = jnp.concatenate(
            [xf, jnp.zeros((Bp - B, Cin, H, W), jnp.float32)], axis=0)

    xt = jnp.transpose(xf, (1, 2, 3, 0))

    wb = jnp.concatenate(
        [w1.reshape(-1), w2.reshape(-1), w3.reshape(-1), w4.reshape(-1),
         b1, b2, b3, b4]).astype(jnp.float32)

    fn = functools.partial(_msfeat_kernel,
                           C=C, Cin=Cin, H=H, W=W, Bb=Bb, H2=H2)

    ot = pl.pallas_call(
        fn,
        out_shape=jax.ShapeDtypeStruct((4 * C, H, W, Bp), jnp.float32),
        grid_spec=pltpu.PrefetchScalarGridSpec(
            num_scalar_prefetch=1,
            grid=(NB, 2),
            in_specs=[
                pl.BlockSpec((Cin, H, W, Bb), lambda i, j, *_: (0, 0, 0, i)),
            ],
            out_specs=pl.BlockSpec((4 * C, H2, W, Bb),
                                   lambda i, j, *_: (0, j, 0, i)),
            scratch_shapes=[pltpu.VMEM((2 * C, H + 4, W, Bb), jnp.float32)],
        ),
        compiler_params=pltpu.CompilerParams(
            dimension_semantics=("parallel", "arbitrary"),
            vmem_limit_bytes=64 << 20),
    )(wb, xt)

    out = jnp.transpose(ot, (3, 0, 1, 2))
    return out[:B] if Bp != B else out


# speedup vs baseline: 7.9343x; 1.0114x over previous
import functools

import jax
import jax.numpy as jnp
from jax.experimental import pallas as pl
from jax.experimental.pallas import tpu as pltpu

_LANE = 128


def _msfeat_kernel(wb_ref,
                   x_ref,
                   o_ref,
                   cs_ref,
                   *, C, Cin, H, W, Bb):
    j = pl.program_id(1)
    nw = 9 * C * (2 * Cin + 2 * C)
    n1 = 9 * C * Cin
    n3 = 9 * C * C

    def wshift(p, dx):
        if dx == 0:
            return p
        z = jnp.zeros((H, abs(dx), Bb), jnp.float32)
        if dx > 0:
            return jnp.concatenate([p[:, dx:, :], z], axis=1)
        return jnp.concatenate([z, p[:, :W + dx, :]], axis=1)

    def hwin(p, dy):
        if dy == 0:
            return p
        z = jnp.zeros((abs(dy), W, Bb), jnp.float32)
        if dy > 0:
            return jnp.concatenate([p[dy:], z], axis=0)
        return jnp.concatenate([z, p[:H + dy]], axis=0)

    def direct_conv(woff, boff, cbase, dil):
        for oc in range(C):
            Q = [None, None, None]
            for ic in range(Cin):
                xv = x_ref[ic]
                for ky in range(3):
                    win = hwin(xv, dil * (ky - 1))
                    for kx in range(3):
                        t = win * wb_ref[woff + (oc * Cin + ic) * 9
                                         + ky * 3 + kx]
                        Q[kx] = t if Q[kx] is None else Q[kx] + t
            acc = (Q[1] + wb_ref[nw + boff + oc]) \
                + wshift(Q[0], -dil) + wshift(Q[2], dil)
            c = jnp.maximum(acc, 0.0)
            o_ref[oc] = c
            cs_ref[cbase + oc, 2:2 + H] = c
            cs_ref[cbase + oc, 0:2] = jnp.zeros((2, W, Bb), jnp.float32)
            cs_ref[cbase + oc, 2 + H:4 + H] = jnp.zeros((2, W, Bb),
                                                        jnp.float32)

    def chained_conv(src, woff, boff, dil):
        for oc in range(C):
            Q = [None, None, None]
            for ic in range(C):
                for ky in range(3):
                    dy = dil * (ky - 1)
                    win = cs_ref[src + ic, 2 + dy:2 + dy + H]
                    for kx in range(3):
                        t = win * wb_ref[woff + (oc * C + ic) * 9
                                         + ky * 3 + kx]
                        Q[kx] = t if Q[kx] is None else Q[kx] + t
            acc = (Q[1] + wb_ref[nw + boff + oc]) \
                + wshift(Q[0], -dil) + wshift(Q[2], dil)
            o_ref[oc] = jnp.maximum(acc, 0.0)

    @pl.when(j == 0)
    def _c1():
        direct_conv(0, 0 * C, 0 * C, 1)

    @pl.when(j == 1)
    def _c2():
        direct_conv(n1, 1 * C, 1 * C, 2)

    @pl.when(j == 2)
    def _c3():
        chained_conv(1 * C, 2 * n1, 2 * C, 1)

    @pl.when(j == 3)
    def _c4():
        chained_conv(0 * C, 2 * n1 + n3, 3 * C, 2)


def kernel(x, w1, b1, w2, b2, w3, b3, w4, b4):
    B, Cin, H, W = x.shape
    C = w1.shape[0]

    Bb = _LANE
    NB = -(-B // Bb)
    Bp = NB * Bb

    xf = x.astype(jnp.float32)
    if Bp != B:
        xf = jnp.concatenate(
            [xf, jnp.zeros((Bp - B, Cin, H, W), jnp.float32)], axis=0)

    xt = jnp.transpose(xf, (1, 2, 3, 0))

    wb = jnp.concatenate(
        [w1.reshape(-1), w2.reshape(-1), w3.reshape(-1), w4.reshape(-1),
         b1, b2, b3, b4]).astype(jnp.float32)

    fn = functools.partial(_msfeat_kernel, C=C, Cin=Cin, H=H, W=W, Bb=Bb)

    ot = pl.pallas_call(
        fn,
        out_shape=jax.ShapeDtypeStruct((4 * C, H, W, Bp), jnp.float32),
        grid_spec=pltpu.PrefetchScalarGridSpec(
            num_scalar_prefetch=1,
            grid=(NB, 4),
            in_specs=[
                pl.BlockSpec((Cin, H, W, Bb), lambda i, j, *_: (0, 0, 0, i)),
            ],
            out_specs=pl.BlockSpec((C, H, W, Bb),
                                   lambda i, j, *_: (j, 0, 0, i)),
            scratch_shapes=[pltpu.VMEM((2 * C, H + 4, W, Bb), jnp.float32)],
        ),
        compiler_params=pltpu.CompilerParams(
            dimension_semantics=("parallel", "arbitrary"),
            vmem_limit_bytes=64 << 20),
    )(wb, xt)

    out = jnp.transpose(ot, (3, 0, 1, 2))
    return out[:B] if Bp != B else out
